# Initial kernel scaffold; baseline (speedup 1.0000x reference)
import jax
import jax.numpy as jnp
from jax.experimental import pallas as pl


def kernel(pred, label):
    raise NotImplementedError("write your pallas kernel here")


# speedup vs baseline: 202.1235x; 202.1235x over previous
import functools

import jax
import jax.numpy as jnp
from jax import lax
from jax.experimental import pallas as pl
from jax.experimental.pallas import tpu as pltpu
from jax.experimental.pallas import tpu_sc as plsc

_C = 21
_B = 8
_PIX = 512 * 512
_NW = 32
_PER_W = _B * _PIX // _NW
_CHUNK = 16384
_NCH = _PER_W // _CHUNK
_LANES = 16
_HBINS = _C * _LANES


def _sc_hist_body(pred_hbm, label_hbm, out_hbm, pbuf, lbuf, hp, hl, hb):
    wid = lax.axis_index("s") * 2 + lax.axis_index("c")
    base = wid * _PER_W

    zeros = jnp.zeros((_LANES,), jnp.float32)
    for c in range(_C):
        hp[pl.ds(c * _LANES, _LANES)] = zeros
        hl[pl.ds(c * _LANES, _LANES)] = zeros
        hb[pl.ds(c * _LANES, _LANES)] = zeros

    lane = lax.iota(jnp.int32, _LANES)
    ones = jnp.ones((_LANES,), jnp.float32)

    for ch in range(_NCH):
        off = base + ch * _CHUNK
        pltpu.sync_copy(pred_hbm.at[pl.ds(off, _CHUNK)], pbuf)
        pltpu.sync_copy(label_hbm.at[pl.ds(off, _CHUNK)], lbuf)

        def body(i, carry):
            for u in range(8):
                s = pl.ds((i * 8 + u) * _LANES, _LANES)
                p = pbuf[s]
                l = lbuf[s]
                pi = lax.shift_left(p, 4) + lane
                li = lax.shift_left(l, 4) + lane
                plsc.addupdate_scatter(hp, [pi], ones)
                plsc.addupdate_scatter(hl, [li], ones)
                plsc.addupdate_scatter(hb, [pi], ones, mask=p == l)
            return carry

        lax.fori_loop(0, _CHUNK // (8 * _LANES), body, 0)

    obase = wid * 3 * _HBINS
    pltpu.sync_copy(hp, out_hbm.at[pl.ds(obase, _HBINS)])
    pltpu.sync_copy(hl, out_hbm.at[pl.ds(obase + _HBINS, _HBINS)])
    pltpu.sync_copy(hb, out_hbm.at[pl.ds(obase + 2 * _HBINS, _HBINS)])


@functools.cache
def _sc_hist():
    return pl.kernel(
        _sc_hist_body,
        out_type=jax.ShapeDtypeStruct((_NW * 3 * _HBINS,), jnp.float32),
        mesh=plsc.VectorSubcoreMesh(core_axis_name="c", subcore_axis_name="s"),
        compiler_params=pltpu.CompilerParams(needs_layout_passes=False),
        scratch_types=[
            pltpu.VMEM((_CHUNK,), jnp.int32),
            pltpu.VMEM((_CHUNK,), jnp.int32),
            pltpu.VMEM((_HBINS,), jnp.float32),
            pltpu.VMEM((_HBINS,), jnp.float32),
            pltpu.VMEM((_HBINS,), jnp.float32),
        ],
    )


_ROW = 4 * 3 * _HBINS


def _combine_body(parts_ref, out_ref):
    x = parts_ref[...]
    k = lax.broadcasted_iota(jnp.int32, (_ROW, 3 * _C), 0)
    m = lax.broadcasted_iota(jnp.int32, (_ROW, 3 * _C), 1)
    sel = ((k % (3 * _HBINS)) // _HBINS == m // _C) & (
        (k % _HBINS) // _LANES == m % _C
    )
    y = jnp.dot(x, sel.astype(jnp.float32),
                preferred_element_type=jnp.float32)
    cp = y[:, 0:_C]
    cl = y[:, _C:2 * _C]
    cb = y[:, 2 * _C:3 * _C]
    dice = (2.0 * cb) / (cp + cl + 1e-10)
    out_ref[...] = jnp.mean(dice, axis=0, keepdims=True)


def kernel(pred, label):
    pred_flat = pred.reshape(_B * _PIX)
    label_flat = label.reshape(_B * _PIX)
    parts = _sc_hist()(pred_flat, label_flat)
    parts2 = parts.reshape(_B, _ROW)
    out = pl.pallas_call(
        _combine_body,
        out_shape=jax.ShapeDtypeStruct((1, _C), jnp.float32),
    )(parts2)
    return out.reshape(_C)
